# Initial kernel scaffold; baseline (speedup 1.0000x reference)
import jax
import jax.numpy as jnp
from jax.experimental import pallas as pl


def kernel(weeks, age, baseline_fvc, pid, sex_id, smk_id, W_bw, W_ba, alpha_tab, gamma_tab, sex_tab, smk_tab, W_cat):
    raise NotImplementedError("write your pallas kernel here")


# speedup vs baseline: 4.9421x; 4.9421x over previous
import jax
import jax.numpy as jnp
from jax import lax
from jax.experimental import pallas as pl
from jax.experimental.pallas import tpu as pltpu
from jax.experimental.pallas import tpu_sc as plsc

N_PATIENTS = 100000
EMB_DIM = 64
B = 16384
NC = 2
NS = 16
NW = NC * NS
BPW = B // NW
CHUNK = 128
NCH = BPW // CHUNK
L = 16


def _permute(x, idx):
    return lax.gather(
        x, idx[:, None],
        lax.GatherDimensionNumbers(offset_dims=(), collapsed_slice_dims=(0,),
                                   start_index_map=(0,)),
        slice_sizes=(1,), mode=lax.GatherScatterMode.PROMISE_IN_BOUNDS)


def _body(weeks_h, age_h, fvc_h, pid_h, sx_h, sk_h, alpha_h, gamma_h,
          sex_h, smk_h, wc_h, scal_h, out_h,
          idx_v, al_v, gm_v, wk_v, ag_v, fv_v, sx_v, sk_v, out_v,
          st_v, sm_v, wc_v, scal_v, comb_v, sem):
    wid = lax.axis_index("s") * NC + lax.axis_index("c")
    base = wid * BPW

    pltpu.sync_copy(pid_h.at[wid], idx_v)
    copies = []
    for j in range(NCH):
        copies.append(pltpu.async_copy(
            alpha_h.at[idx_v.at[j]], al_v.at[pl.ds(j * CHUNK, CHUNK)], sem))
        copies.append(pltpu.async_copy(
            gamma_h.at[idx_v.at[j]], gm_v.at[pl.ds(j * CHUNK, CHUNK)], sem))

    pltpu.sync_copy(weeks_h.at[pl.ds(base, BPW)], wk_v)
    pltpu.sync_copy(age_h.at[pl.ds(base, BPW)], ag_v)
    pltpu.sync_copy(fvc_h.at[pl.ds(base, BPW)], fv_v)
    pltpu.sync_copy(sx_h.at[pl.ds(base, BPW)], sx_v)
    pltpu.sync_copy(sk_h.at[pl.ds(base, BPW)], sk_v)
    pltpu.sync_copy(sex_h, st_v)
    pltpu.sync_copy(smk_h, sm_v)
    pltpu.sync_copy(wc_h, wc_v)
    pltpu.sync_copy(scal_h, scal_v)

    lanes = lax.broadcasted_iota(jnp.int32, (L,), 0)

    def dot_row(tab_ref, row, off):
        acc = tab_ref[row, pl.ds(0, L)] * wc_v[pl.ds(off, L)]
        for t in range(1, EMB_DIM // L):
            acc = acc + tab_ref[row, pl.ds(t * L, L)] * wc_v[pl.ds(off + t * L, L)]
        for sh in (8, 4, 2, 1):
            acc = acc + _permute(acc, lanes ^ sh)
        return acc

    sexp = [dot_row(st_v, j, 0) for j in range(3)]
    smkp = [dot_row(sm_v, m, EMB_DIM) for m in range(4)]

    jj = lanes // 4
    mm = lanes - jj * 4
    sexv = jnp.where(jj == 0, sexp[0], jnp.where(jj == 1, sexp[1], sexp[2]))
    smkv = jnp.where(mm == 0, smkp[0],
                     jnp.where(mm == 1, smkp[1],
                               jnp.where(mm == 2, smkp[2], smkp[3])))
    comb_v[...] = sexv + smkv

    wbwv = scal_v[pl.ds(0, L)]
    wbav = scal_v[pl.ds(L, L)]

    for c in copies:
        c.wait()

    for g in range(BPW // L):
        sl = pl.ds(g * L, L)
        w = wk_v[sl]
        cid = sx_v[sl] * 4 + sk_v[sl]
        catv = plsc.load_gather(comb_v, [cid])
        out_v[sl] = (fv_v[sl] + w * wbwv + ag_v[sl] * wbav
                     + al_v[sl] + gm_v[sl] * w + catv)

    pltpu.sync_copy(out_v, out_h.at[pl.ds(base, BPW)])


def kernel(weeks, age, baseline_fvc, pid, sex_id, smk_id,
           W_bw, W_ba, alpha_tab, gamma_tab, sex_tab, smk_tab, W_cat):
    f32 = jnp.float32
    weeks1 = weeks.reshape(B)
    age1 = age.reshape(B)
    fvc1 = baseline_fvc.reshape(B)
    pid3 = pid.astype(jnp.int32).reshape(NW, NCH, CHUNK)
    sx1 = sex_id.astype(jnp.int32).reshape(B)
    sk1 = smk_id.astype(jnp.int32).reshape(B)
    alpha1 = alpha_tab.reshape(N_PATIENTS)
    gamma1 = gamma_tab.reshape(N_PATIENTS)
    wc1 = W_cat.reshape(2 * EMB_DIM)
    scal = jnp.concatenate([jnp.broadcast_to(W_bw.reshape(1), (L,)),
                            jnp.broadcast_to(W_ba.reshape(1), (L,))])

    mesh = plsc.VectorSubcoreMesh(core_axis_name="c", subcore_axis_name="s",
                                  num_cores=NC, num_subcores=NS)
    run = pl.kernel(
        _body,
        out_type=jax.ShapeDtypeStruct((B,), f32),
        mesh=mesh,
        compiler_params=pltpu.CompilerParams(needs_layout_passes=False),
        scratch_types=[
            pltpu.VMEM((NCH, CHUNK), jnp.int32),
            pltpu.VMEM((BPW,), f32),
            pltpu.VMEM((BPW,), f32),
            pltpu.VMEM((BPW,), f32),
            pltpu.VMEM((BPW,), f32),
            pltpu.VMEM((BPW,), f32),
            pltpu.VMEM((BPW,), jnp.int32),
            pltpu.VMEM((BPW,), jnp.int32),
            pltpu.VMEM((BPW,), f32),
            pltpu.VMEM((3, EMB_DIM), f32),
            pltpu.VMEM((4, EMB_DIM), f32),
            pltpu.VMEM((2 * EMB_DIM,), f32),
            pltpu.VMEM((2 * L,), f32),
            pltpu.VMEM((L,), f32),
            pltpu.SemaphoreType.DMA,
        ],
    )
    return run(weeks1, age1, fvc1, pid3, sx1, sk1, alpha1, gamma1,
               sex_tab, smk_tab, wc1, scal)


# speedup vs baseline: 5.9669x; 1.2074x over previous
import jax
import jax.numpy as jnp
from jax import lax
from jax.experimental import pallas as pl
from jax.experimental.pallas import tpu as pltpu
from jax.experimental.pallas import tpu_sc as plsc

N_PATIENTS = 100000
EMB_DIM = 64
B = 16384
NC = 2
NS = 16
NW = NC * NS
BPW = B // NW
CHUNK = 128
NCH = BPW // CHUNK
L = 16


def _permute(x, idx):
    return lax.gather(
        x, idx[:, None],
        lax.GatherDimensionNumbers(offset_dims=(), collapsed_slice_dims=(0,),
                                   start_index_map=(0,)),
        slice_sizes=(1,), mode=lax.GatherScatterMode.PROMISE_IN_BOUNDS)


def _body(weeks_h, age_h, fvc_h, pid_h, sx_h, sk_h, wbw_h, wba_h,
          alpha_h, gamma_h, sex_h, smk_h, wc_h, out_h,
          idx_v, al_v, gm_v, wk_v, ag_v, fv_v, sx_v, sk_v, out_v,
          st_v, sm_v, wc_v, wbw_v, wba_v, comb_v, gsem, ssem):
    wid = lax.axis_index("s") * NC + lax.axis_index("c")
    base = wid * BPW
    sl_w = pl.ds(base, BPW)

    pltpu.sync_copy(pid_h.at[wid], idx_v)
    gathers = []
    for j in range(NCH):
        gathers.append(pltpu.async_copy(
            alpha_h.at[idx_v.at[j]], al_v.at[pl.ds(j * CHUNK, CHUNK)], gsem))
        gathers.append(pltpu.async_copy(
            gamma_h.at[idx_v.at[j]], gm_v.at[pl.ds(j * CHUNK, CHUNK)], gsem))
    stages = [
        pltpu.async_copy(sex_h, st_v, ssem),
        pltpu.async_copy(smk_h, sm_v, ssem),
        pltpu.async_copy(wc_h, wc_v, ssem),
        pltpu.async_copy(wbw_h, wbw_v, ssem),
        pltpu.async_copy(wba_h, wba_v, ssem),
        pltpu.async_copy(weeks_h.at[sl_w], wk_v, ssem),
        pltpu.async_copy(age_h.at[sl_w], ag_v, ssem),
        pltpu.async_copy(fvc_h.at[sl_w], fv_v, ssem),
        pltpu.async_copy(sx_h.at[sl_w], sx_v, ssem),
        pltpu.async_copy(sk_h.at[sl_w], sk_v, ssem),
    ]
    for c in stages:
        c.wait()

    lanes = lax.broadcasted_iota(jnp.int32, (L,), 0)
    zeros = lanes - lanes

    def dot_row(tab_ref, row, off):
        acc = tab_ref[row, pl.ds(0, L)] * wc_v[pl.ds(off, L)]
        for t in range(1, EMB_DIM // L):
            acc = acc + tab_ref[row, pl.ds(t * L, L)] * wc_v[pl.ds(off + t * L, L)]
        for sh in (8, 4, 2, 1):
            acc = acc + _permute(acc, lanes ^ sh)
        return acc

    sexp = [dot_row(st_v, j, 0) for j in range(3)]
    smkp = [dot_row(sm_v, m, EMB_DIM) for m in range(4)]

    jj = lanes // 4
    mm = lanes - jj * 4
    sexv = jnp.where(jj == 0, sexp[0], jnp.where(jj == 1, sexp[1], sexp[2]))
    smkv = jnp.where(mm == 0, smkp[0],
                     jnp.where(mm == 1, smkp[1],
                               jnp.where(mm == 2, smkp[2], smkp[3])))
    comb_v[...] = sexv + smkv

    wbw = plsc.load_gather(wbw_v, [zeros])
    wba = plsc.load_gather(wba_v, [zeros])

    for c in gathers:
        c.wait()

    for g in range(BPW // L):
        sl = pl.ds(g * L, L)
        w = wk_v[sl]
        cid = sx_v[sl] * 4 + sk_v[sl]
        catv = plsc.load_gather(comb_v, [cid])
        out_v[sl] = (fv_v[sl] + w * wbw + ag_v[sl] * wba
                     + al_v[sl] + gm_v[sl] * w + catv)

    pltpu.sync_copy(out_v, out_h.at[sl_w])


def kernel(weeks, age, baseline_fvc, pid, sex_id, smk_id,
           W_bw, W_ba, alpha_tab, gamma_tab, sex_tab, smk_tab, W_cat):
    f32 = jnp.float32
    weeks1 = weeks.reshape(B)
    age1 = age.reshape(B)
    fvc1 = baseline_fvc.reshape(B)
    pid3 = pid.astype(jnp.int32).reshape(NW, NCH, CHUNK)
    sx1 = sex_id.astype(jnp.int32).reshape(B)
    sk1 = smk_id.astype(jnp.int32).reshape(B)
    alpha1 = alpha_tab.reshape(N_PATIENTS)
    gamma1 = gamma_tab.reshape(N_PATIENTS)
    wc1 = W_cat.reshape(2 * EMB_DIM)
    wbw1 = W_bw.reshape(1)
    wba1 = W_ba.reshape(1)

    mesh = plsc.VectorSubcoreMesh(core_axis_name="c", subcore_axis_name="s",
                                  num_cores=NC, num_subcores=NS)
    run = pl.kernel(
        _body,
        out_type=jax.ShapeDtypeStruct((B,), f32),
        mesh=mesh,
        compiler_params=pltpu.CompilerParams(needs_layout_passes=False),
        scratch_types=[
            pltpu.VMEM((NCH, CHUNK), jnp.int32),
            pltpu.VMEM((BPW,), f32),
            pltpu.VMEM((BPW,), f32),
            pltpu.VMEM((BPW,), f32),
            pltpu.VMEM((BPW,), f32),
            pltpu.VMEM((BPW,), f32),
            pltpu.VMEM((BPW,), jnp.int32),
            pltpu.VMEM((BPW,), jnp.int32),
            pltpu.VMEM((BPW,), f32),
            pltpu.VMEM((3, EMB_DIM), f32),
            pltpu.VMEM((4, EMB_DIM), f32),
            pltpu.VMEM((2 * EMB_DIM,), f32),
            pltpu.VMEM((1,), f32),
            pltpu.VMEM((1,), f32),
            pltpu.VMEM((L,), f32),
            pltpu.SemaphoreType.DMA,
            pltpu.SemaphoreType.DMA,
        ],
    )
    return run(weeks1, age1, fvc1, pid3, sx1, sk1, wbw1, wba1,
               alpha1, gamma1, sex_tab, smk_tab, wc1)


# speedup vs baseline: 5.9887x; 1.0037x over previous
import jax
import jax.numpy as jnp
from jax import lax
from jax.experimental import pallas as pl
from jax.experimental.pallas import tpu as pltpu
from jax.experimental.pallas import tpu_sc as plsc

N_PATIENTS = 100000
EMB_DIM = 64
B = 16384
NC = 2
NS = 16
NW = NC * NS
BPW = B // NW
CHUNK = 128
NCH = BPW // CHUNK
L = 16


def _permute(x, idx):
    return lax.gather(
        x, idx[:, None],
        lax.GatherDimensionNumbers(offset_dims=(), collapsed_slice_dims=(0,),
                                   start_index_map=(0,)),
        slice_sizes=(1,), mode=lax.GatherScatterMode.PROMISE_IN_BOUNDS)


def _body(weeks_h, age_h, fvc_h, pid_h, sx_h, sk_h, wbw_h, wba_h,
          alpha_h, gamma_h, sex_h, smk_h, wc_h, out_h,
          idx_v, al_v, gm_v, wk_v, ag_v, fv_v, sx_v, sk_v, out_v,
          st_v, sm_v, wc_v, wbw_v, wba_v, comb_v, gsem, ssem):
    wid = lax.axis_index("s") * NC + lax.axis_index("c")
    base = wid * BPW
    sl_w = pl.ds(base, BPW)

    pltpu.sync_copy(pid_h.at[wid], idx_v)
    gathers = []
    for j in range(NCH):
        gathers.append(pltpu.async_copy(
            alpha_h.at[idx_v.at[j]], al_v.at[pl.ds(j * CHUNK, CHUNK)], gsem))
        gathers.append(pltpu.async_copy(
            gamma_h.at[idx_v.at[j]], gm_v.at[pl.ds(j * CHUNK, CHUNK)], gsem))
    stages = [
        pltpu.async_copy(sex_h, st_v, ssem),
        pltpu.async_copy(smk_h, sm_v, ssem),
        pltpu.async_copy(wc_h, wc_v, ssem),
        pltpu.async_copy(wbw_h, wbw_v, ssem),
        pltpu.async_copy(wba_h, wba_v, ssem),
        pltpu.async_copy(weeks_h.at[sl_w], wk_v, ssem),
        pltpu.async_copy(age_h.at[sl_w], ag_v, ssem),
        pltpu.async_copy(fvc_h.at[sl_w], fv_v, ssem),
        pltpu.async_copy(sx_h.at[sl_w], sx_v, ssem),
        pltpu.async_copy(sk_h.at[sl_w], sk_v, ssem),
    ]
    for c in stages:
        c.wait()

    lanes = lax.broadcasted_iota(jnp.int32, (L,), 0)
    zeros = lanes - lanes

    def dot_row(tab_ref, row, off):
        acc = tab_ref[row, pl.ds(0, L)] * wc_v[pl.ds(off, L)]
        for t in range(1, EMB_DIM // L):
            acc = acc + tab_ref[row, pl.ds(t * L, L)] * wc_v[pl.ds(off + t * L, L)]
        return jnp.sum(acc)

    sexp = [dot_row(st_v, j, 0) for j in range(3)]
    smkp = [dot_row(sm_v, m, EMB_DIM) for m in range(4)]

    jj = lanes // 4
    mm = lanes - jj * 4
    sexv = jnp.where(jj == 0, sexp[0], jnp.where(jj == 1, sexp[1], sexp[2]))
    smkv = jnp.where(mm == 0, smkp[0],
                     jnp.where(mm == 1, smkp[1],
                               jnp.where(mm == 2, smkp[2], smkp[3])))
    comb_v[...] = sexv + smkv

    wbw = plsc.load_gather(wbw_v, [zeros])
    wba = plsc.load_gather(wba_v, [zeros])

    for c in gathers:
        c.wait()

    def combine(g, carry):
        sl = pl.ds(g * L, L)
        w = wk_v[sl]
        cid = sx_v[sl] * 4 + sk_v[sl]
        catv = plsc.load_gather(comb_v, [cid])
        out_v[sl] = (fv_v[sl] + w * wbw + ag_v[sl] * wba
                     + al_v[sl] + gm_v[sl] * w + catv)
        return carry

    lax.fori_loop(0, BPW // L, combine, 0, unroll=4)

    pltpu.sync_copy(out_v, out_h.at[sl_w])


def kernel(weeks, age, baseline_fvc, pid, sex_id, smk_id,
           W_bw, W_ba, alpha_tab, gamma_tab, sex_tab, smk_tab, W_cat):
    f32 = jnp.float32
    weeks1 = weeks.reshape(B)
    age1 = age.reshape(B)
    fvc1 = baseline_fvc.reshape(B)
    pid3 = pid.astype(jnp.int32).reshape(NW, NCH, CHUNK)
    sx1 = sex_id.astype(jnp.int32).reshape(B)
    sk1 = smk_id.astype(jnp.int32).reshape(B)
    alpha1 = alpha_tab.reshape(N_PATIENTS)
    gamma1 = gamma_tab.reshape(N_PATIENTS)
    wc1 = W_cat.reshape(2 * EMB_DIM)
    wbw1 = W_bw.reshape(1)
    wba1 = W_ba.reshape(1)

    mesh = plsc.VectorSubcoreMesh(core_axis_name="c", subcore_axis_name="s",
                                  num_cores=NC, num_subcores=NS)
    run = pl.kernel(
        _body,
        out_type=jax.ShapeDtypeStruct((B,), f32),
        mesh=mesh,
        compiler_params=pltpu.CompilerParams(needs_layout_passes=False),
        scratch_types=[
            pltpu.VMEM((NCH, CHUNK), jnp.int32),
            pltpu.VMEM((BPW,), f32),
            pltpu.VMEM((BPW,), f32),
            pltpu.VMEM((BPW,), f32),
            pltpu.VMEM((BPW,), f32),
            pltpu.VMEM((BPW,), f32),
            pltpu.VMEM((BPW,), jnp.int32),
            pltpu.VMEM((BPW,), jnp.int32),
            pltpu.VMEM((BPW,), f32),
            pltpu.VMEM((3, EMB_DIM), f32),
            pltpu.VMEM((4, EMB_DIM), f32),
            pltpu.VMEM((2 * EMB_DIM,), f32),
            pltpu.VMEM((1,), f32),
            pltpu.VMEM((1,), f32),
            pltpu.VMEM((L,), f32),
            pltpu.SemaphoreType.DMA,
            pltpu.SemaphoreType.DMA,
        ],
    )
    return run(weeks1, age1, fvc1, pid3, sx1, sk1, wbw1, wba1,
               alpha1, gamma1, sex_tab, smk_tab, wc1)


# speedup vs baseline: 6.0774x; 1.0148x over previous
import jax
import jax.numpy as jnp
from jax import lax
from jax.experimental import pallas as pl
from jax.experimental.pallas import tpu as pltpu
from jax.experimental.pallas import tpu_sc as plsc

N_PATIENTS = 100000
EMB_DIM = 64
B = 16384
NC = 2
NS = 16
NW = NC * NS
BPW = B // NW
CHUNK = 128
NCH = BPW // CHUNK
L = 16


def _permute(x, idx):
    return lax.gather(
        x, idx[:, None],
        lax.GatherDimensionNumbers(offset_dims=(), collapsed_slice_dims=(0,),
                                   start_index_map=(0,)),
        slice_sizes=(1,), mode=lax.GatherScatterMode.PROMISE_IN_BOUNDS)


def _body(weeks_h, age_h, fvc_h, pid_h, sx_h, sk_h, wbw_h, wba_h,
          alpha_h, gamma_h, sex_h, smk_h, wc_h, out_h,
          idx_v, al_v, gm_v, wk_v, ag_v, fv_v, sx_v, sk_v, out_v,
          st_v, sm_v, wc_v, wbw_v, wba_v, comb_v, gsem, ssem):
    wid = lax.axis_index("s") * NC + lax.axis_index("c")
    base = wid * BPW
    sl_w = pl.ds(base, BPW)

    pltpu.sync_copy(pid_h.at[wid], idx_v)
    gathers = []
    for j in range(NCH):
        gathers.append(pltpu.async_copy(
            alpha_h.at[idx_v.at[j]], al_v.at[pl.ds(j * CHUNK, CHUNK)], gsem))
        gathers.append(pltpu.async_copy(
            gamma_h.at[idx_v.at[j]], gm_v.at[pl.ds(j * CHUNK, CHUNK)], gsem))
    stages = [
        pltpu.async_copy(sex_h, st_v, ssem),
        pltpu.async_copy(smk_h, sm_v, ssem),
        pltpu.async_copy(wc_h, wc_v, ssem),
        pltpu.async_copy(wbw_h, wbw_v, ssem),
        pltpu.async_copy(wba_h, wba_v, ssem),
        pltpu.async_copy(weeks_h.at[sl_w], wk_v, ssem),
        pltpu.async_copy(age_h.at[sl_w], ag_v, ssem),
        pltpu.async_copy(fvc_h.at[sl_w], fv_v, ssem),
        pltpu.async_copy(sx_h.at[sl_w], sx_v, ssem),
        pltpu.async_copy(sk_h.at[sl_w], sk_v, ssem),
    ]

    lanes = lax.broadcasted_iota(jnp.int32, (L,), 0)
    zeros = lanes - lanes

    for c in stages:
        c.wait()

    def dot_row(tab_ref, row, off):
        acc = tab_ref[row, pl.ds(0, L)] * wc_v[pl.ds(off, L)]
        for t in range(1, EMB_DIM // L):
            acc = acc + tab_ref[row, pl.ds(t * L, L)] * wc_v[pl.ds(off + t * L, L)]
        return jnp.sum(acc)

    sexp = [dot_row(st_v, j, 0) for j in range(3)]
    smkp = [dot_row(sm_v, m, EMB_DIM) for m in range(4)]

    jj = lanes // 4
    mm = lanes - jj * 4
    sexv = jnp.where(jj == 0, sexp[0], jnp.where(jj == 1, sexp[1], sexp[2]))
    smkv = jnp.where(mm == 0, smkp[0],
                     jnp.where(mm == 1, smkp[1],
                               jnp.where(mm == 2, smkp[2], smkp[3])))
    comb_v[...] = sexv + smkv

    wbw = plsc.load_gather(wbw_v, [zeros])
    wba = plsc.load_gather(wba_v, [zeros])

    def dense_pass(g, carry):
        sl = pl.ds(g * L, L)
        cid = sx_v[sl] * 4 + sk_v[sl]
        catv = plsc.load_gather(comb_v, [cid])
        out_v[sl] = (fv_v[sl] + wk_v[sl] * wbw + ag_v[sl] * wba + catv)
        return carry

    lax.fori_loop(0, BPW // L, dense_pass, 0, unroll=4)

    for c in gathers:
        c.wait()

    def patient_pass(g, carry):
        sl = pl.ds(g * L, L)
        out_v[sl] = out_v[sl] + al_v[sl] + gm_v[sl] * wk_v[sl]
        return carry

    lax.fori_loop(0, BPW // L, patient_pass, 0, unroll=4)

    pltpu.sync_copy(out_v, out_h.at[sl_w])


def kernel(weeks, age, baseline_fvc, pid, sex_id, smk_id,
           W_bw, W_ba, alpha_tab, gamma_tab, sex_tab, smk_tab, W_cat):
    f32 = jnp.float32
    weeks1 = weeks.reshape(B)
    age1 = age.reshape(B)
    fvc1 = baseline_fvc.reshape(B)
    pid3 = pid.astype(jnp.int32).reshape(NW, NCH, CHUNK)
    sx1 = sex_id.astype(jnp.int32).reshape(B)
    sk1 = smk_id.astype(jnp.int32).reshape(B)
    alpha1 = alpha_tab.reshape(N_PATIENTS)
    gamma1 = gamma_tab.reshape(N_PATIENTS)
    wc1 = W_cat.reshape(2 * EMB_DIM)
    wbw1 = W_bw.reshape(1)
    wba1 = W_ba.reshape(1)

    mesh = plsc.VectorSubcoreMesh(core_axis_name="c", subcore_axis_name="s",
                                  num_cores=NC, num_subcores=NS)
    run = pl.kernel(
        _body,
        out_type=jax.ShapeDtypeStruct((B,), f32),
        mesh=mesh,
        compiler_params=pltpu.CompilerParams(needs_layout_passes=False),
        scratch_types=[
            pltpu.VMEM((NCH, CHUNK), jnp.int32),
            pltpu.VMEM((BPW,), f32),
            pltpu.VMEM((BPW,), f32),
            pltpu.VMEM((BPW,), f32),
            pltpu.VMEM((BPW,), f32),
            pltpu.VMEM((BPW,), f32),
            pltpu.VMEM((BPW,), jnp.int32),
            pltpu.VMEM((BPW,), jnp.int32),
            pltpu.VMEM((BPW,), f32),
            pltpu.VMEM((3, EMB_DIM), f32),
            pltpu.VMEM((4, EMB_DIM), f32),
            pltpu.VMEM((2 * EMB_DIM,), f32),
            pltpu.VMEM((1,), f32),
            pltpu.VMEM((1,), f32),
            pltpu.VMEM((L,), f32),
            pltpu.SemaphoreType.DMA,
            pltpu.SemaphoreType.DMA,
        ],
    )
    return run(weeks1, age1, fvc1, pid3, sx1, sk1, wbw1, wba1,
               alpha1, gamma1, sex_tab, smk_tab, wc1)
